# Initial kernel scaffold; baseline (speedup 1.0000x reference)
import jax
import jax.numpy as jnp
from jax.experimental import pallas as pl


def kernel(x, edge_index, W1, b1, W2, b2, W3, b3, Wl, bl):
    raise NotImplementedError("write your pallas kernel here")


# speedup vs baseline: 27.1806x; 27.1806x over previous
import functools

import jax
import jax.numpy as jnp
from jax import lax
from jax.experimental import pallas as pl
from jax.experimental.pallas import tpu as pltpu
from jax.experimental.pallas import tpu_sc as plsc

N = 10000
NP = 10240
E = 320000
EP = 327680
NC = 2
NS = 16
EW = EP // (NC * NS)
CH = 128
NCH = EW // CH
SLAB = NP // NS

f32 = jnp.float32
i32 = jnp.int32

_mesh = plsc.VectorSubcoreMesh(core_axis_name="c", subcore_axis_name="s")


def _fill(buf, rows, cols, value):
    v = jnp.full((16,), value, f32)
    for r in range(rows):
        for j in range(cols // 16):
            buf[r, pl.ds(j * 16, 16)] = v


@functools.partial(
    pl.kernel,
    out_type=jax.ShapeDtypeStruct((NC, NP, 16), f32),
    mesh=_mesh,
    scratch_types=[
        pltpu.VMEM_SHARED((NP, 16), f32),
        pltpu.VMEM((16, 16), f32),
        pltpu.VMEM((CH, 16), f32),
        pltpu.VMEM((CH,), i32),
    ],
)
def _sc_degree(dst_hbm, out_hbm, acc, zbuf, obuf, sidx):
    cid = lax.axis_index("c")
    sid = lax.axis_index("s")
    _fill(zbuf, 16, 16, 0.0)
    _fill(obuf, CH, 16, 1.0)
    base = sid * SLAB

    def zrow(i, carry):
        pltpu.sync_copy(zbuf, acc.at[pl.ds(base + i * 16, 16)])
        return carry

    lax.fori_loop(0, SLAB // 16, zrow, None)
    plsc.subcore_barrier()

    ebase = (cid * NS + sid) * EW

    def body(g, carry):
        pltpu.sync_copy(dst_hbm.at[pl.ds(ebase + g * CH, CH)], sidx)
        pltpu.sync_copy(obuf, acc.at[sidx], add=True)
        return carry

    lax.fori_loop(0, NCH, body, None)
    plsc.subcore_barrier()
    pltpu.sync_copy(acc.at[pl.ds(base, SLAB)], out_hbm.at[cid, pl.ds(base, SLAB)])


@functools.partial(
    pl.kernel,
    out_type=jax.ShapeDtypeStruct((NC, NP, 128), f32),
    mesh=_mesh,
    scratch_types=[
        pltpu.VMEM_SHARED((NP, 128), f32),
        pltpu.VMEM((16, 128), f32),
        pltpu.VMEM((CH, 128), f32),
        pltpu.VMEM((CH,), i32),
        pltpu.VMEM((CH,), i32),
        pltpu.SemaphoreType.DMA,
    ],
)
def _sc_agg128(xs_hbm, src_hbm, dst_hbm, out_hbm, acc, zbuf, gbuf, gidx, sidx,
               sem):
    cid = lax.axis_index("c")
    sid = lax.axis_index("s")
    _fill(zbuf, 16, 128, 0.0)
    base = sid * SLAB

    def zrow(i, carry):
        pltpu.sync_copy(zbuf, acc.at[pl.ds(base + i * 16, 16)])
        return carry

    lax.fori_loop(0, SLAB // 16, zrow, None)
    plsc.subcore_barrier()

    ebase = (cid * NS + sid) * EW

    def body(g, carry):
        off = ebase + g * CH
        pltpu.sync_copy(src_hbm.at[pl.ds(off, CH)], gidx)
        pltpu.sync_copy(dst_hbm.at[pl.ds(off, CH)], sidx)
        pltpu.async_copy(xs_hbm.at[gidx], gbuf, sem).wait()
        pltpu.sync_copy(gbuf, acc.at[sidx], add=True)
        return carry

    lax.fori_loop(0, NCH, body, None)
    plsc.subcore_barrier()
    pltpu.sync_copy(acc.at[pl.ds(base, SLAB)], out_hbm.at[cid, pl.ds(base, SLAB)])


@functools.partial(
    pl.kernel,
    out_type=(
        jax.ShapeDtypeStruct((NC, NP, 16), f32),
        jax.ShapeDtypeStruct((NC, NP, 16), f32),
    ),
    mesh=_mesh,
    scratch_types=[
        pltpu.VMEM_SHARED((NP, 16), f32),
        pltpu.VMEM_SHARED((NP, 16), f32),
        pltpu.VMEM_SHARED((NP, 16), f32),
        pltpu.VMEM_SHARED((NP, 16), f32),
        pltpu.VMEM((16, 16), f32),
        pltpu.VMEM((CH, 16), f32),
        pltpu.VMEM((CH, 16), f32),
        pltpu.VMEM((CH,), i32),
        pltpu.VMEM((CH,), i32),
        pltpu.SemaphoreType.DMA,
        pltpu.SemaphoreType.DMA,
    ],
)
def _sc_pass2(z2s_hbm, dinv_hbm, src_hbm, dst_hbm, outa_hbm, outs_hbm,
              ztab, dtab, acca, accs, zbuf, bufa, bufs, gidx, sidx,
              sema, sems):
    cid = lax.axis_index("c")
    sid = lax.axis_index("s")
    _fill(zbuf, 16, 16, 0.0)
    base = sid * SLAB

    pltpu.sync_copy(z2s_hbm.at[pl.ds(base, SLAB)], ztab.at[pl.ds(base, SLAB)])
    pltpu.sync_copy(dinv_hbm.at[pl.ds(base, SLAB)], dtab.at[pl.ds(base, SLAB)])

    def zrow(i, carry):
        pltpu.sync_copy(zbuf, acca.at[pl.ds(base + i * 16, 16)])
        pltpu.sync_copy(zbuf, accs.at[pl.ds(base + i * 16, 16)])
        return carry

    lax.fori_loop(0, SLAB // 16, zrow, None)
    plsc.subcore_barrier()

    ebase = (cid * NS + sid) * EW

    def body(g, carry):
        off = ebase + g * CH
        pltpu.sync_copy(src_hbm.at[pl.ds(off, CH)], gidx)
        pltpu.sync_copy(dst_hbm.at[pl.ds(off, CH)], sidx)
        cpa = pltpu.async_copy(ztab.at[gidx], bufa, sema)
        cps = pltpu.async_copy(dtab.at[sidx], bufs, sems)
        cpa.wait()
        cps.wait()
        pltpu.sync_copy(bufa, acca.at[sidx], add=True)
        pltpu.sync_copy(bufs, accs.at[gidx], add=True)
        return carry

    lax.fori_loop(0, NCH, body, None)
    plsc.subcore_barrier()
    pltpu.sync_copy(acca.at[pl.ds(base, SLAB)],
                    outa_hbm.at[cid, pl.ds(base, SLAB)])
    pltpu.sync_copy(accs.at[pl.ds(base, SLAB)],
                    outs_hbm.at[cid, pl.ds(base, SLAB)])


RB = 1280


def _tc_prep(d0, d1, xp):
    def body(d0_ref, d1_ref, x_ref, xs_ref, dv_ref):
        deg = d0_ref[...] + d1_ref[...] + 1.0
        dinv = lax.rsqrt(deg)
        dv_ref[...] = dinv
        xs_ref[...] = x_ref[...] * dinv[:, 0:1]

    return pl.pallas_call(
        body,
        grid=(NP // RB,),
        in_specs=[
            pl.BlockSpec((RB, 16), lambda i: (i, 0)),
            pl.BlockSpec((RB, 16), lambda i: (i, 0)),
            pl.BlockSpec((RB, 128), lambda i: (i, 0)),
        ],
        out_specs=[
            pl.BlockSpec((RB, 128), lambda i: (i, 0)),
            pl.BlockSpec((RB, 16), lambda i: (i, 0)),
        ],
        out_shape=[
            jax.ShapeDtypeStruct((NP, 128), f32),
            jax.ShapeDtypeStruct((NP, 16), f32),
        ],
    )(d0, d1, xp)


RM = 512


def _tc_mid(r0, r1, xs, dv, w1, b1r, w2):
    def body(r0_ref, r1_ref, xs_ref, dv_ref, w1_ref, b1_ref, w2_ref, out_ref):
        dinv = dv_ref[:, 0:1]
        a = (r0_ref[...] + r1_ref[...] + xs_ref[...]) * dinv
        h1 = jnp.dot(a, w1_ref[...], preferred_element_type=f32) + b1_ref[...]
        h1 = jnp.maximum(h1, 0.0)
        z2 = jnp.dot(h1, w2_ref[...], preferred_element_type=f32)
        out_ref[...] = z2 * dinv

    return pl.pallas_call(
        body,
        grid=(NP // RM,),
        in_specs=[
            pl.BlockSpec((RM, 128), lambda i: (i, 0)),
            pl.BlockSpec((RM, 128), lambda i: (i, 0)),
            pl.BlockSpec((RM, 128), lambda i: (i, 0)),
            pl.BlockSpec((RM, 16), lambda i: (i, 0)),
            pl.BlockSpec((128, 1024), lambda i: (0, 0)),
            pl.BlockSpec((1, 1024), lambda i: (0, 0)),
            pl.BlockSpec((1024, 16), lambda i: (0, 0)),
        ],
        out_specs=pl.BlockSpec((RM, 16), lambda i: (i, 0)),
        out_shape=jax.ShapeDtypeStruct((NP, 16), f32),
    )(r0, r1, xs, dv, w1, b1r, w2)


RF = 1280


def _tc_final(r0, r1, z2s, dv, s0, s1, b2r, w3, b3r, wlp, blp):
    def body(r0_ref, r1_ref, z_ref, dv_ref, s0_ref, s1_ref, b2_ref, w3_ref,
             b3_ref, wl_ref, bl_ref, out_ref, acc):
        i = pl.program_id(0)
        dinv = dv_ref[:, 0:1]
        h2 = (r0_ref[...] + r1_ref[...] + z_ref[...]) * dinv + b2_ref[...]
        h2 = jnp.maximum(h2, 0.0)
        s = s0_ref[:, 0:1] + s1_ref[:, 0:1]
        c = dinv * s + dinv * dinv
        rows = i * RF + lax.broadcasted_iota(i32, (RF, 1), 0)
        c = jnp.where(rows < N, c, 0.0)
        part = jnp.sum(h2 * c, axis=0, keepdims=True)

        @pl.when(i == 0)
        def _():
            acc[...] = jnp.zeros_like(acc)

        acc[...] += part

        @pl.when(i == NP // RF - 1)
        def _():
            pooled = acc[...] * (1.0 / N)
            t = jnp.dot(pooled, w3_ref[...], preferred_element_type=f32)
            t = t + b3_ref[...]
            out_ref[...] = (
                jnp.dot(t, wl_ref[...], preferred_element_type=f32) + bl_ref[...]
            )

    return pl.pallas_call(
        body,
        grid=(NP // RF,),
        in_specs=[
            pl.BlockSpec((RF, 16), lambda i: (i, 0)),
            pl.BlockSpec((RF, 16), lambda i: (i, 0)),
            pl.BlockSpec((RF, 16), lambda i: (i, 0)),
            pl.BlockSpec((RF, 16), lambda i: (i, 0)),
            pl.BlockSpec((RF, 16), lambda i: (i, 0)),
            pl.BlockSpec((RF, 16), lambda i: (i, 0)),
            pl.BlockSpec((1, 16), lambda i: (0, 0)),
            pl.BlockSpec((16, 16), lambda i: (0, 0)),
            pl.BlockSpec((1, 16), lambda i: (0, 0)),
            pl.BlockSpec((16, 128), lambda i: (0, 0)),
            pl.BlockSpec((1, 128), lambda i: (0, 0)),
        ],
        out_specs=pl.BlockSpec((1, 128), lambda i: (0, 0)),
        out_shape=jax.ShapeDtypeStruct((1, 128), f32),
        scratch_shapes=[pltpu.VMEM((1, 16), f32)],
    )(r0, r1, z2s, dv, s0, s1, b2r, w3, b3r, wlp, blp)


def kernel(x, edge_index, W1, b1, W2, b2, W3, b3, Wl, bl):
    src = edge_index[0].astype(i32)
    dst = edge_index[1].astype(i32)
    pad_e = N + (jnp.arange(EP - E, dtype=i32) % (NP - N))
    src_p = jnp.concatenate([src, pad_e])
    dst_p = jnp.concatenate([dst, pad_e])
    x_p = jnp.pad(x, ((0, NP - N), (0, 0)))

    deg2 = _sc_degree(dst_p)
    xs, dinv16 = _tc_prep(deg2[0], deg2[1], x_p)
    agg1 = _sc_agg128(xs, src_p, dst_p)
    z2s = _tc_mid(agg1[0], agg1[1], xs, dinv16, W1,
                  b1.reshape(1, 1024), W2)
    agg2, s16 = _sc_pass2(z2s, dinv16, src_p, dst_p)
    wlp = jnp.pad(Wl, ((0, 0), (0, 125)))
    blp = jnp.pad(bl, (0, 125)).reshape(1, 128)
    out = _tc_final(agg2[0], agg2[1], z2s, dinv16, s16[0], s16[1],
                    b2.reshape(1, 16), W3, b3.reshape(1, 16), wlp, blp)
    return out[:, :3]
